# baseline (speedup 1.0000x reference)
import functools

import jax
import jax.numpy as jnp
from jax import lax
from jax.experimental import pallas as pl
from jax.experimental.pallas import tpu as pltpu
from jax.experimental.pallas import tpu_sc as plsc

NUM_BINS = 256
N_PIX = 512 * 512
N_CH = 16 * 3
LANES = 16
CH_PER_TILE = 3
CHUNK = 16384
N_CHUNKS = N_PIX // CHUNK
VREGS = CHUNK // LANES


def _hist_body(pred_hbm, target_hbm, out_hbm, buf0, buf1, hist, outbuf,
               sem0, sem1):
    cid = lax.axis_index("c")
    sid = lax.axis_index("s")
    wid = sid * 2 + cid
    is_pred = wid < 16
    local = lax.rem(wid, 16)
    lane = lax.iota(jnp.int32, LANES)
    lane_k = [lane | (k << 12) for k in range(4)]
    ones = jnp.ones((LANES,), jnp.float32)
    zeros = jnp.zeros((LANES,), jnp.float32)
    bufs = (buf0, buf1)
    sems = (sem0, sem1)

    def start_copy(off, b):
        @pl.when(is_pred)
        def _():
            pltpu.async_copy(pred_hbm.at[pl.ds(off, CHUNK)], bufs[b], sems[b])

        @pl.when(jnp.logical_not(is_pred))
        def _():
            pltpu.async_copy(target_hbm.at[pl.ds(off, CHUNK)], bufs[b],
                             sems[b])

    def wait_copy(b):
        pltpu.make_async_copy(pred_hbm.at[pl.ds(0, CHUNK)], bufs[b],
                              sems[b]).wait()

    for j in range(CH_PER_TILE):
        ch = local * CH_PER_TILE + j
        base = ch * N_PIX

        def zero_body(k, carry):
            hist[pl.ds(k * LANES, LANES)] = zeros
            return carry

        lax.fori_loop(0, 4 * NUM_BINS, zero_body, 0, unroll=8)

        start_copy(base, 0)

        def pair_body(i2, carry):
            for b in range(2):
                cidx = i2 * 2 + b

                @pl.when(cidx + 1 < N_CHUNKS)
                def _():
                    start_copy(base + (cidx + 1) * CHUNK, 1 - b)

                wait_copy(b)
                buf = bufs[b]

                def vec_body(i, c2):
                    for u in range(8):
                        x = buf[pl.ds((i * 8 + u) * LANES, LANES)]
                        off = (x * 4096.0).astype(jnp.int32)
                        off = (off & 0xFF0) | lane_k[u % 4]
                        plsc.addupdate_scatter(hist, [off], ones)
                    return c2

                lax.fori_loop(0, VREGS // 8, vec_body, 0)
            return carry

        lax.fori_loop(0, N_CHUNKS // 2, pair_body, 0)

        def fold_body(m, carry):
            b = m * LANES
            v = (hist[pl.ds(b, LANES)] + hist[pl.ds(4096 + b, LANES)]
                 + hist[pl.ds(8192 + b, LANES)]
                 + hist[pl.ds(12288 + b, LANES)])
            hist[pl.ds(b, LANES)] = v
            return carry

        lax.fori_loop(0, NUM_BINS, fold_body, 0, unroll=4)

        def red_body(g, carry):
            row = hist[pl.ds(g * LANES, LANES)]
            s = jnp.sum(row)
            plsc.store_scatter(outbuf, [jnp.broadcast_to(g, (LANES,))],
                               jnp.broadcast_to(s, (LANES,)),
                               mask=lane == 0)
            return carry

        lax.fori_loop(0, NUM_BINS, red_body, 0, unroll=4)

        u_row = jnp.where(is_pred, ch, N_CH + ch)
        pltpu.sync_copy(outbuf, out_hbm.at[pl.ds(u_row * NUM_BINS, NUM_BINS)])


@functools.partial(
    pl.kernel,
    mesh=plsc.VectorSubcoreMesh(core_axis_name="c", subcore_axis_name="s"),
    out_type=jax.ShapeDtypeStruct((2 * N_CH * NUM_BINS,), jnp.float32),
    scratch_types=[
        pltpu.VMEM((CHUNK,), jnp.float32),
        pltpu.VMEM((CHUNK,), jnp.float32),
        pltpu.VMEM((4 * NUM_BINS * LANES,), jnp.float32),
        pltpu.VMEM((NUM_BINS,), jnp.float32),
        pltpu.SemaphoreType.DMA,
        pltpu.SemaphoreType.DMA,
    ],
    compiler_params=pltpu.CompilerParams(needs_layout_passes=False),
)
def _hist_kernel(pred_hbm, target_hbm, out_hbm, buf0, buf1, hist, outbuf,
                 sem0, sem1):
    _hist_body(pred_hbm, target_hbm, out_hbm, buf0, buf1, hist, outbuf,
               sem0, sem1)


def kernel(pred, target):
    hist = _hist_kernel(pred.reshape(-1), target.reshape(-1))
    hist = hist.reshape(2, N_CH, NUM_BINS)
    p = hist[0] / (hist[0].sum(axis=1, keepdims=True) + 1e-8)
    t = hist[1] / (hist[1].sum(axis=1, keepdims=True) + 1e-8)
    return jnp.mean((p - t) ** 2)


# speedup vs baseline: 3.5680x; 3.5680x over previous
import functools

import jax
import jax.numpy as jnp
from jax import lax
from jax.experimental import pallas as pl
from jax.experimental.pallas import tpu as pltpu
from jax.experimental.pallas import tpu_sc as plsc

NUM_BINS = 256
N_PIX = 512 * 512
N_CH = 16 * 3
LANES = 16
CH_PER_TILE = 3
CHUNK = 16384
N_CHUNKS = N_PIX // CHUNK
VREGS = CHUNK // LANES


def _hist_body(pred_hbm, target_hbm, out_hbm, buf0, buf1, hist, outbuf,
               sem0, sem1):
    cid = lax.axis_index("c")
    sid = lax.axis_index("s")
    wid = sid * 2 + cid
    is_pred = wid < 16
    local = lax.rem(wid, 16)
    lane = lax.iota(jnp.int32, LANES)
    lane_k = [lane | (k << 12) for k in range(4)]
    ones = jnp.ones((LANES,), jnp.float32)
    zeros = jnp.zeros((LANES,), jnp.float32)
    bufs = (buf0, buf1)
    sems = (sem0, sem1)

    def start_copy(off, b):
        @pl.when(is_pred)
        def _():
            pltpu.async_copy(pred_hbm.at[pl.ds(off, CHUNK)], bufs[b], sems[b])

        @pl.when(jnp.logical_not(is_pred))
        def _():
            pltpu.async_copy(target_hbm.at[pl.ds(off, CHUNK)], bufs[b],
                             sems[b])

    def wait_copy(b):
        pltpu.make_async_copy(pred_hbm.at[pl.ds(0, CHUNK)], bufs[b],
                              sems[b]).wait()

    for j in range(CH_PER_TILE):
        ch = local * CH_PER_TILE + j
        base = ch * N_PIX

        def zero_body(k, carry):
            hist[pl.ds(k * LANES, LANES)] = zeros
            return carry

        lax.fori_loop(0, 4 * NUM_BINS, zero_body, 0, unroll=8)

        start_copy(base, 0)

        def pair_body(i2, carry):
            for b in range(2):
                cidx = i2 * 2 + b

                @pl.when(cidx + 1 < N_CHUNKS)
                def _():
                    start_copy(base + (cidx + 1) * CHUNK, 1 - b)

                wait_copy(b)
                buf = bufs[b]

                @plsc.parallel_loop(0, VREGS, 1, unroll=8)
                def _(i):
                    x = buf[pl.ds(i * LANES, LANES)]
                    off = (x * 4096.0).astype(jnp.int32)
                    off = (off & 0xFF0) | lane
                    plsc.addupdate_scatter(hist, [off], ones)
            return carry

        lax.fori_loop(0, N_CHUNKS // 2, pair_body, 0)

        def fold_body(m, carry):
            b = m * LANES
            v = (hist[pl.ds(b, LANES)] + hist[pl.ds(4096 + b, LANES)]
                 + hist[pl.ds(8192 + b, LANES)]
                 + hist[pl.ds(12288 + b, LANES)])
            hist[pl.ds(b, LANES)] = v
            return carry

        lax.fori_loop(0, NUM_BINS, fold_body, 0, unroll=4)

        def red_body(g, carry):
            row = hist[pl.ds(g * LANES, LANES)]
            s = jnp.sum(row)
            plsc.store_scatter(outbuf, [jnp.broadcast_to(g, (LANES,))],
                               jnp.broadcast_to(s, (LANES,)),
                               mask=lane == 0)
            return carry

        lax.fori_loop(0, NUM_BINS, red_body, 0, unroll=4)

        u_row = jnp.where(is_pred, ch, N_CH + ch)
        pltpu.sync_copy(outbuf, out_hbm.at[pl.ds(u_row * NUM_BINS, NUM_BINS)])


@functools.partial(
    pl.kernel,
    mesh=plsc.VectorSubcoreMesh(core_axis_name="c", subcore_axis_name="s"),
    out_type=jax.ShapeDtypeStruct((2 * N_CH * NUM_BINS,), jnp.float32),
    scratch_types=[
        pltpu.VMEM((CHUNK,), jnp.float32),
        pltpu.VMEM((CHUNK,), jnp.float32),
        pltpu.VMEM((4 * NUM_BINS * LANES,), jnp.float32),
        pltpu.VMEM((NUM_BINS,), jnp.float32),
        pltpu.SemaphoreType.DMA,
        pltpu.SemaphoreType.DMA,
    ],
    compiler_params=pltpu.CompilerParams(needs_layout_passes=False),
)
def _hist_kernel(pred_hbm, target_hbm, out_hbm, buf0, buf1, hist, outbuf,
                 sem0, sem1):
    _hist_body(pred_hbm, target_hbm, out_hbm, buf0, buf1, hist, outbuf,
               sem0, sem1)


def kernel(pred, target):
    hist = _hist_kernel(pred.reshape(-1), target.reshape(-1))
    hist = hist.reshape(2, N_CH, NUM_BINS)
    p = hist[0] / (hist[0].sum(axis=1, keepdims=True) + 1e-8)
    t = hist[1] / (hist[1].sum(axis=1, keepdims=True) + 1e-8)
    return jnp.mean((p - t) ** 2)


# speedup vs baseline: 6.9049x; 1.9352x over previous
import functools

import jax
import jax.numpy as jnp
from jax import lax
from jax.experimental import pallas as pl
from jax.experimental.pallas import tpu as pltpu
from jax.experimental.pallas import tpu_sc as plsc

NUM_BINS = 256
IMG_B, IMG_C, IMG_H, IMG_W = 16, 3, 512, 512
N_CH = IMG_B * IMG_C
LANES = 16
UNITS_PER_TILE = 3
HALF_ROWS = IMG_H // 2
ROWS = 32
CHUNK = ROWS * IMG_W
N_CHUNKS = HALF_ROWS // ROWS
VREGS = CHUNK // LANES
HSIZE = NUM_BINS * LANES


def _hist_body(pred_hbm, target_hbm, out_hbm, buf0, buf1, hist, outbuf,
               sem0, sem1):
    cid = lax.axis_index("c")
    sid = lax.axis_index("s")
    wid = sid * 2 + cid
    lane = lax.iota(jnp.int32, LANES)
    lane_t = lane | (1 << 12)
    ones = jnp.ones((LANES,), jnp.float32)
    zeros = jnp.zeros((LANES,), jnp.float32)
    bufs = (buf0, buf1)
    sems = (sem0, sem1)

    def start_copy(src, row0, b):
        pltpu.async_copy(src.at[pl.ds(row0, ROWS), :], bufs[b], sems[b])

    def wait_copy(b):
        pltpu.make_async_copy(
            pred_hbm.at[pl.ds(0, ROWS), :], bufs[b], sems[b]).wait()

    for j in range(UNITS_PER_TILE):
        u = wid * UNITS_PER_TILE + j
        rbase = u * HALF_ROWS

        @plsc.parallel_loop(0, 2 * NUM_BINS, 1, unroll=8)
        def _(k):
            hist[pl.ds(k * LANES, LANES)] = zeros

        start_copy(pred_hbm, rbase, 0)

        def pair_body(s2, carry):
            for b in range(2):
                s = s2 * 2 + b

                @pl.when(s + 1 < 2 * N_CHUNKS)
                def _():
                    if b == 0:
                        start_copy(target_hbm, rbase + s2 * ROWS, 1)
                    else:
                        start_copy(pred_hbm, rbase + (s2 + 1) * ROWS, 0)

                wait_copy(b)
                buf = bufs[b]
                lane_c = lane if b == 0 else lane_t

                @plsc.parallel_loop(0, VREGS, 1, unroll=8)
                def _(i):
                    x = buf[i >> 5, pl.ds((i & 31) * LANES, LANES)]
                    off = (x * 4096.0).astype(jnp.int32)
                    off = (off & 0xFF0) | lane_c
                    plsc.addupdate_scatter(hist, [off], ones)
            return carry

        lax.fori_loop(0, N_CHUNKS, pair_body, 0)

        @plsc.parallel_loop(0, 2 * NUM_BINS, 1, unroll=4)
        def _(g):
            row = hist[pl.ds(g * LANES, LANES)]
            s = jnp.sum(row)
            plsc.store_scatter(outbuf, [jnp.broadcast_to(g, (LANES,))],
                               jnp.broadcast_to(s, (LANES,)),
                               mask=lane == 0)

        pltpu.sync_copy(outbuf,
                        out_hbm.at[pl.ds(u * 2 * NUM_BINS, 2 * NUM_BINS)])


@functools.partial(
    pl.kernel,
    mesh=plsc.VectorSubcoreMesh(core_axis_name="c", subcore_axis_name="s"),
    out_type=jax.ShapeDtypeStruct((96 * 2 * NUM_BINS,), jnp.float32),
    scratch_types=[
        pltpu.VMEM((ROWS, IMG_W), jnp.float32),
        pltpu.VMEM((ROWS, IMG_W), jnp.float32),
        pltpu.VMEM((2 * HSIZE,), jnp.float32),
        pltpu.VMEM((2 * NUM_BINS,), jnp.float32),
        pltpu.SemaphoreType.DMA,
        pltpu.SemaphoreType.DMA,
    ],
    compiler_params=pltpu.CompilerParams(needs_layout_passes=False),
)
def _hist_kernel(pred_hbm, target_hbm, out_hbm, buf0, buf1, hist, outbuf,
                 sem0, sem1):
    _hist_body(pred_hbm, target_hbm, out_hbm, buf0, buf1, hist, outbuf,
               sem0, sem1)


def kernel(pred, target):
    part = _hist_kernel(pred.reshape(N_CH * IMG_H, IMG_W),
                        target.reshape(N_CH * IMG_H, IMG_W))
    part = part.reshape(N_CH, 2, 2, NUM_BINS).sum(axis=1)
    p = part[:, 0, :]
    t = part[:, 1, :]
    p = p / (p.sum(axis=1, keepdims=True) + 1e-8)
    t = t / (t.sum(axis=1, keepdims=True) + 1e-8)
    return jnp.mean((p - t) ** 2)
